# baseline (speedup 1.0000x reference)
import jax
import jax.numpy as jnp
from jax import lax
from jax.experimental import pallas as pl

_N = 320000
_D = 128
_M = 10000
_H = 128

_BT = 1280
_NBT = _N // _BT
_W = 128
_MP = 10120


def _seg_body(x_ref, ids_ref, o_ref):
  i = pl.program_id(0)

  @pl.when(i == 0)
  def _():
    o_ref[...] = jnp.zeros((_MP, _D), jnp.float32)

  ids = ids_ref[...].reshape(1, _BT)

  def onehot_mm(loc, valid):
    oh = (lax.broadcasted_iota(jnp.int32, (_W, _BT), 0) == loc)
    oh = jnp.logical_and(oh, valid)
    return jnp.dot(oh.astype(jnp.float32), x_ref[...],
                   preferred_element_type=jnp.float32)

  w0 = jnp.bitwise_and(jnp.min(ids), -8)
  loc = ids - w0
  part = onehot_mm(loc, jnp.full((1, _BT), True))
  o_ref[pl.ds(w0, _W), :] = o_ref[pl.ds(w0, _W), :] + part

  def cond(carry):
    return jnp.max(carry) > 0

  def body(carry):
    sel = carry > 0
    w1 = jnp.bitwise_and(jnp.min(jnp.where(sel, ids, jnp.int32(2 ** 30))),
                         -8)
    loc2 = ids - w1
    part2 = onehot_mm(loc2, sel)
    o_ref[pl.ds(w1, _W), :] = o_ref[pl.ds(w1, _W), :] + part2
    return carry * (loc2 >= _W).astype(jnp.int32)

  lax.while_loop(cond, body, (loc >= _W).astype(jnp.int32))


_segment_sum = pl.pallas_call(
    _seg_body,
    grid=(_NBT,),
    in_specs=[
        pl.BlockSpec((_BT, _D), lambda i: (i, 0)),
        pl.BlockSpec((1, 1, _BT), lambda i: (i, 0, 0)),
    ],
    out_specs=pl.BlockSpec((_MP, _D), lambda i: (0, 0)),
    out_shape=jax.ShapeDtypeStruct((_MP, _D), jnp.float32),
)


_BM = 2000


def _mlp_body(agg_ref, w1_ref, b1_ref, w2_ref, b2_ref, o_ref):
  agg = agg_ref[...]
  h = jnp.dot(agg, w1_ref[...].T, preferred_element_type=jnp.float32)
  h = h + b1_ref[...]
  h = h * jax.nn.sigmoid(h)
  y = jnp.sum(h * w2_ref[...], axis=1, keepdims=True) + b2_ref[...]
  o_ref[...] = y


_mlp = pl.pallas_call(
    _mlp_body,
    grid=(_M // _BM,),
    in_specs=[
        pl.BlockSpec((_BM, _D), lambda i: (i, 0)),
        pl.BlockSpec((_H, _D), lambda i: (0, 0)),
        pl.BlockSpec((1, _H), lambda i: (0, 0)),
        pl.BlockSpec((1, _H), lambda i: (0, 0)),
        pl.BlockSpec((1, 1), lambda i: (0, 0)),
    ],
    out_specs=pl.BlockSpec((_BM, 1), lambda i: (i, 0)),
    out_shape=jax.ShapeDtypeStruct((_M, 1), jnp.float32),
)


def kernel(scalar_representation, idx_m, W1, b1, W2, b2):
  agg = _segment_sum(scalar_representation, idx_m.reshape(_N // _BT, 1, _BT))
  return _mlp(agg, W1, b1.reshape(1, _H), W2, b2.reshape(1, 1))


# speedup vs baseline: 1.1216x; 1.1216x over previous
import jax
import jax.numpy as jnp
from jax import lax
from jax.experimental import pallas as pl

_N = 320000
_D = 128
_M = 10000
_H = 128

_BT = 1280
_NBT = _N // _BT
_W = 64
_MP = 10056


def _seg_body(x_ref, ids_ref, o_ref):
  i = pl.program_id(0)

  @pl.when(i == 0)
  def _():
    o_ref[...] = jnp.zeros((_MP, _D), jnp.float32)

  ids = ids_ref[...].reshape(1, _BT)

  def onehot_mm(loc, valid):
    oh = (lax.broadcasted_iota(jnp.int32, (_W, _BT), 0) == loc)
    oh = jnp.logical_and(oh, valid)
    return jnp.dot(oh.astype(jnp.float32), x_ref[...],
                   preferred_element_type=jnp.float32)

  w0 = jnp.bitwise_and(ids[0, 0], -8)
  loc = ids - w0
  part = onehot_mm(loc, jnp.full((1, _BT), True))
  o_ref[pl.ds(w0, _W), :] = o_ref[pl.ds(w0, _W), :] + part

  def cond(carry):
    return jnp.max(carry) > 0

  def body(carry):
    sel = carry > 0
    w1 = jnp.bitwise_and(jnp.min(jnp.where(sel, ids, jnp.int32(2 ** 30))),
                         -8)
    loc2 = ids - w1
    part2 = onehot_mm(loc2, sel)
    o_ref[pl.ds(w1, _W), :] = o_ref[pl.ds(w1, _W), :] + part2
    return carry * (loc2 >= _W).astype(jnp.int32)

  lax.while_loop(cond, body, (loc >= _W).astype(jnp.int32))


_segment_sum = pl.pallas_call(
    _seg_body,
    grid=(_NBT,),
    in_specs=[
        pl.BlockSpec((_BT, _D), lambda i: (i, 0)),
        pl.BlockSpec((1, 1, _BT), lambda i: (i, 0, 0)),
    ],
    out_specs=pl.BlockSpec((_MP, _D), lambda i: (0, 0)),
    out_shape=jax.ShapeDtypeStruct((_MP, _D), jnp.float32),
)


_BM = 2000


def _mlp_body(agg_ref, w1_ref, b1_ref, w2_ref, b2_ref, o_ref):
  agg = agg_ref[...]
  h = jnp.dot(agg, w1_ref[...].T, preferred_element_type=jnp.float32)
  h = h + b1_ref[...]
  h = h * jax.nn.sigmoid(h)
  y = jnp.sum(h * w2_ref[...], axis=1, keepdims=True) + b2_ref[...]
  o_ref[...] = y


_mlp = pl.pallas_call(
    _mlp_body,
    grid=(_M // _BM,),
    in_specs=[
        pl.BlockSpec((_BM, _D), lambda i: (i, 0)),
        pl.BlockSpec((_H, _D), lambda i: (0, 0)),
        pl.BlockSpec((1, _H), lambda i: (0, 0)),
        pl.BlockSpec((1, _H), lambda i: (0, 0)),
        pl.BlockSpec((1, 1), lambda i: (0, 0)),
    ],
    out_specs=pl.BlockSpec((_BM, 1), lambda i: (i, 0)),
    out_shape=jax.ShapeDtypeStruct((_M, 1), jnp.float32),
)


def kernel(scalar_representation, idx_m, W1, b1, W2, b2):
  agg = _segment_sum(scalar_representation, idx_m.reshape(_N // _BT, 1, _BT))
  return _mlp(agg, W1, b1.reshape(1, _H), W2, b2.reshape(1, 1))


# speedup vs baseline: 1.7071x; 1.5220x over previous
import jax
import jax.numpy as jnp
from jax import lax
from jax.experimental import pallas as pl

_N = 320000
_D = 128
_M = 10000
_H = 128

_BT = 2560
_NBT = _N // _BT
_W = 128
_MP = 10120


def _seg_body(x_ref, ids_ref, o_ref):
  i = pl.program_id(0)

  @pl.when(i == 0)
  def _():
    o_ref[...] = jnp.zeros((_MP, _D), jnp.float32)

  ids = ids_ref[...].reshape(1, _BT)

  def onehot_mm(loc, valid):
    oh = (lax.broadcasted_iota(jnp.int32, (_W, _BT), 0) == loc)
    oh = jnp.logical_and(oh, valid)
    return jnp.dot(oh.astype(jnp.float32), x_ref[...],
                   preferred_element_type=jnp.float32)

  w0 = jnp.bitwise_and(ids[0, 0], -8)
  loc = ids - w0
  part = onehot_mm(loc, jnp.full((1, _BT), True))
  o_ref[pl.ds(w0, _W), :] = o_ref[pl.ds(w0, _W), :] + part

  def cond(carry):
    return jnp.max(carry) > 0

  def body(carry):
    sel = carry > 0
    w1 = jnp.bitwise_and(jnp.min(jnp.where(sel, ids, jnp.int32(2 ** 30))),
                         -8)
    loc2 = ids - w1
    part2 = onehot_mm(loc2, sel)
    o_ref[pl.ds(w1, _W), :] = o_ref[pl.ds(w1, _W), :] + part2
    return carry * (loc2 >= _W).astype(jnp.int32)

  lax.while_loop(cond, body, (loc >= _W).astype(jnp.int32))


_segment_sum = pl.pallas_call(
    _seg_body,
    grid=(_NBT,),
    in_specs=[
        pl.BlockSpec((_BT, _D), lambda i: (i, 0)),
        pl.BlockSpec((1, 1, _BT), lambda i: (i, 0, 0)),
    ],
    out_specs=pl.BlockSpec((_MP, _D), lambda i: (0, 0)),
    out_shape=jax.ShapeDtypeStruct((_MP, _D), jnp.float32),
)


_BM = 2000


def _mlp_body(agg_ref, w1_ref, b1_ref, w2_ref, b2_ref, o_ref):
  agg = agg_ref[...]
  h = jnp.dot(agg, w1_ref[...].T, preferred_element_type=jnp.float32)
  h = h + b1_ref[...]
  h = h * jax.nn.sigmoid(h)
  y = jnp.sum(h * w2_ref[...], axis=1, keepdims=True) + b2_ref[...]
  o_ref[...] = y


_mlp = pl.pallas_call(
    _mlp_body,
    grid=(_M // _BM,),
    in_specs=[
        pl.BlockSpec((_BM, _D), lambda i: (i, 0)),
        pl.BlockSpec((_H, _D), lambda i: (0, 0)),
        pl.BlockSpec((1, _H), lambda i: (0, 0)),
        pl.BlockSpec((1, _H), lambda i: (0, 0)),
        pl.BlockSpec((1, 1), lambda i: (0, 0)),
    ],
    out_specs=pl.BlockSpec((_BM, 1), lambda i: (i, 0)),
    out_shape=jax.ShapeDtypeStruct((_M, 1), jnp.float32),
)


def kernel(scalar_representation, idx_m, W1, b1, W2, b2):
  agg = _segment_sum(scalar_representation, idx_m.reshape(_N // _BT, 1, _BT))
  return _mlp(agg, W1, b1.reshape(1, _H), W2, b2.reshape(1, 1))


# speedup vs baseline: 1.9166x; 1.1227x over previous
import jax
import jax.numpy as jnp
from jax import lax
from jax.experimental import pallas as pl

_N = 320000
_D = 128
_M = 10000
_H = 128

_BT = 3200
_NBT = _N // _BT
_W = 128
_MP = 10120


def _seg_body(x_ref, ids_ref, o_ref):
  i = pl.program_id(0)

  @pl.when(i == 0)
  def _():
    o_ref[...] = jnp.zeros((_MP, _D), jnp.float32)

  ids = ids_ref[...].reshape(1, _BT)

  def onehot_mm(loc, valid):
    oh = (lax.broadcasted_iota(jnp.int32, (_W, _BT), 0) == loc)
    oh = jnp.logical_and(oh, valid)
    return jnp.dot(oh.astype(jnp.float32), x_ref[...],
                   preferred_element_type=jnp.float32)

  w0 = jnp.bitwise_and(ids[0, 0], -8)
  loc = ids - w0
  part = onehot_mm(loc, jnp.full((1, _BT), True))
  o_ref[pl.ds(w0, _W), :] = o_ref[pl.ds(w0, _W), :] + part

  def cond(carry):
    return jnp.max(carry) > 0

  def body(carry):
    sel = carry > 0
    w1 = jnp.bitwise_and(jnp.min(jnp.where(sel, ids, jnp.int32(2 ** 30))),
                         -8)
    loc2 = ids - w1
    part2 = onehot_mm(loc2, sel)
    o_ref[pl.ds(w1, _W), :] = o_ref[pl.ds(w1, _W), :] + part2
    return carry * (loc2 >= _W).astype(jnp.int32)

  lax.while_loop(cond, body, (loc >= _W).astype(jnp.int32))


_segment_sum = pl.pallas_call(
    _seg_body,
    grid=(_NBT,),
    in_specs=[
        pl.BlockSpec((_BT, _D), lambda i: (i, 0)),
        pl.BlockSpec((1, 1, _BT), lambda i: (i, 0, 0)),
    ],
    out_specs=pl.BlockSpec((_MP, _D), lambda i: (0, 0)),
    out_shape=jax.ShapeDtypeStruct((_MP, _D), jnp.float32),
)


_BM = 2000


def _mlp_body(agg_ref, w1_ref, b1_ref, w2_ref, b2_ref, o_ref):
  agg = agg_ref[...]
  h = jnp.dot(agg, w1_ref[...].T, preferred_element_type=jnp.float32)
  h = h + b1_ref[...]
  h = h * jax.nn.sigmoid(h)
  y = jnp.sum(h * w2_ref[...], axis=1, keepdims=True) + b2_ref[...]
  o_ref[...] = y


_mlp = pl.pallas_call(
    _mlp_body,
    grid=(_M // _BM,),
    in_specs=[
        pl.BlockSpec((_BM, _D), lambda i: (i, 0)),
        pl.BlockSpec((_H, _D), lambda i: (0, 0)),
        pl.BlockSpec((1, _H), lambda i: (0, 0)),
        pl.BlockSpec((1, _H), lambda i: (0, 0)),
        pl.BlockSpec((1, 1), lambda i: (0, 0)),
    ],
    out_specs=pl.BlockSpec((_BM, 1), lambda i: (i, 0)),
    out_shape=jax.ShapeDtypeStruct((_M, 1), jnp.float32),
)


def kernel(scalar_representation, idx_m, W1, b1, W2, b2):
  agg = _segment_sum(scalar_representation, idx_m.reshape(_N // _BT, 1, _BT))
  return _mlp(agg, W1, b1.reshape(1, _H), W2, b2.reshape(1, 1))
